# Initial kernel scaffold; baseline (speedup 1.0000x reference)
import jax
import jax.numpy as jnp
from jax.experimental import pallas as pl


def kernel(inputs, targets):
    raise NotImplementedError("write your pallas kernel here")


# speedup vs baseline: 57.6215x; 57.6215x over previous
import functools

import jax
import jax.numpy as jnp
from jax import lax
from jax.experimental import pallas as pl
from jax.experimental.pallas import tpu as pltpu
from jax.experimental.pallas import tpu_sc as plsc

_NC = 2
_NS = 16
_NW = _NC * _NS
_L = 16

_CHUNK = 16384


def _make_sc_count(n):
    per_worker = n // _NW
    n_chunks = per_worker // _CHUNK
    groups = _CHUNK // _L
    mesh = plsc.VectorSubcoreMesh(core_axis_name="c", subcore_axis_name="s")

    @functools.partial(
        pl.kernel,
        mesh=mesh,
        out_type=jax.ShapeDtypeStruct((_NW, 2, _L), jnp.int32),
        scratch_types=[
            pltpu.VMEM((_CHUNK,), jnp.float32),
            pltpu.VMEM((_CHUNK,), jnp.int32),
            pltpu.VMEM((2, _L), jnp.int32),
        ],
    )
    def sc_count(x_hbm, t_hbm, out_hbm, xbuf, tbuf, accbuf):
        wid = lax.axis_index("s") * _NC + lax.axis_index("c")
        base = wid * per_worker

        def chunk_body(c, carry):
            tn, st = carry
            off = base + c * _CHUNK
            pltpu.sync_copy(x_hbm.at[pl.ds(off, _CHUNK)], xbuf)
            pltpu.sync_copy(t_hbm.at[pl.ds(off, _CHUNK)], tbuf)

            def group_body(g, gcarry):
                gtn, gst = gcarry
                vx = xbuf[pl.ds(g * _L, _L)]
                vt = tbuf[pl.ds(g * _L, _L)]
                m = (vx < 0.5) & (vt == 0)
                gtn = gtn + jnp.where(m, 1, 0)
                gst = gst + vt
                return gtn, gst

            return lax.fori_loop(0, groups, group_body, (tn, st))

        zero = jnp.zeros((_L,), jnp.int32)
        tn, st = lax.fori_loop(0, n_chunks, chunk_body, (zero, zero))
        accbuf[0, :] = tn
        accbuf[1, :] = st
        pltpu.sync_copy(accbuf, out_hbm.at[wid])

    return sc_count


def kernel(inputs, targets):
    n = inputs.shape[0]
    parts = _make_sc_count(n)(inputs, targets)
    tn = parts[:, 0, :].sum()
    st = parts[:, 1, :].sum()
    t0 = n - st
    return tn.astype(jnp.float32) / jnp.clip(t0.astype(jnp.float32), 1e-12)


# speedup vs baseline: 66.9260x; 1.1615x over previous
import functools

import jax
import jax.numpy as jnp
from jax import lax
from jax.experimental import pallas as pl
from jax.experimental.pallas import tpu as pltpu
from jax.experimental.pallas import tpu_sc as plsc

_NC = 2
_NS = 16
_NW = _NC * _NS
_L = 16

_CHUNK = 16384


def _make_sc_count(n):
    per_worker = n // _NW
    n_chunks = per_worker // _CHUNK
    groups = _CHUNK // _L
    mesh = plsc.VectorSubcoreMesh(core_axis_name="c", subcore_axis_name="s")

    @functools.partial(
        pl.kernel,
        mesh=mesh,
        out_type=jax.ShapeDtypeStruct((_NW, 2, _L), jnp.int32),
        scratch_types=[
            pltpu.VMEM((2, _CHUNK), jnp.float32),
            pltpu.VMEM((2, _CHUNK), jnp.int32),
            pltpu.VMEM((2, _L), jnp.int32),
            pltpu.SemaphoreType.DMA,
            pltpu.SemaphoreType.DMA,
        ],
    )
    def sc_count(x_hbm, t_hbm, out_hbm, xbuf, tbuf, accbuf, sem0, sem1):
        wid = lax.axis_index("s") * _NC + lax.axis_index("c")
        base = wid * per_worker
        sems = (sem0, sem1)

        def copies(c, slot):
            off = base + c * _CHUNK
            return (
                pltpu.make_async_copy(
                    x_hbm.at[pl.ds(off, _CHUNK)], xbuf.at[slot], sems[slot]),
                pltpu.make_async_copy(
                    t_hbm.at[pl.ds(off, _CHUNK)], tbuf.at[slot], sems[slot]),
            )

        for cp in copies(0, 0):
            cp.start()

        tn = jnp.zeros((_L,), jnp.int32)
        st = jnp.zeros((_L,), jnp.int32)
        for c in range(n_chunks):
            slot = c % 2
            if c + 1 < n_chunks:
                for cp in copies(c + 1, (c + 1) % 2):
                    cp.start()
            for cp in copies(c, slot):
                cp.wait()

            def group_body(g, gcarry, slot=slot):
                gtn, gst = gcarry
                vx = xbuf[slot, pl.ds(g * _L, _L)]
                vt = tbuf[slot, pl.ds(g * _L, _L)]
                m = (vx < 0.5) & (vt == 0)
                gtn = gtn + jnp.where(m, 1, 0)
                gst = gst + vt
                return gtn, gst

            tn, st = lax.fori_loop(0, groups, group_body, (tn, st))

        accbuf[0, :] = tn
        accbuf[1, :] = st
        pltpu.sync_copy(accbuf, out_hbm.at[wid])

    return sc_count


def kernel(inputs, targets):
    n = inputs.shape[0]
    parts = _make_sc_count(n)(inputs, targets)
    tn = parts[:, 0, :].sum()
    st = parts[:, 1, :].sum()
    t0 = n - st
    return tn.astype(jnp.float32) / jnp.clip(t0.astype(jnp.float32), 1e-12)


# speedup vs baseline: 94.8766x; 1.4176x over previous
import functools

import jax
import jax.numpy as jnp
from jax import lax
from jax.experimental import pallas as pl
from jax.experimental.pallas import tpu as pltpu
from jax.experimental.pallas import tpu_sc as plsc

_NC = 2
_NS = 16
_NW = _NC * _NS
_L = 16

_CHUNK = 16384
_UNROLL = 8


def _make_sc_count(n):
    per_worker = n // _NW
    n_chunks = per_worker // _CHUNK
    groups = _CHUNK // _L
    mesh = plsc.VectorSubcoreMesh(core_axis_name="c", subcore_axis_name="s")

    @functools.partial(
        pl.kernel,
        mesh=mesh,
        out_type=jax.ShapeDtypeStruct((_NW, 2, _L), jnp.int32),
        scratch_types=[
            pltpu.VMEM((2, _CHUNK), jnp.float32),
            pltpu.VMEM((2, _CHUNK), jnp.int32),
            pltpu.VMEM((2, _L), jnp.int32),
            pltpu.SemaphoreType.DMA,
            pltpu.SemaphoreType.DMA,
        ],
    )
    def sc_count(x_hbm, t_hbm, out_hbm, xbuf, tbuf, accbuf, sem0, sem1):
        wid = lax.axis_index("s") * _NC + lax.axis_index("c")
        base = wid * per_worker
        sems = (sem0, sem1)

        def copies(c, slot):
            off = base + c * _CHUNK
            return (
                pltpu.make_async_copy(
                    x_hbm.at[pl.ds(off, _CHUNK)], xbuf.at[slot], sems[slot]),
                pltpu.make_async_copy(
                    t_hbm.at[pl.ds(off, _CHUNK)], tbuf.at[slot], sems[slot]),
            )

        for cp in copies(0, 0):
            cp.start()

        zero = jnp.zeros((_L,), jnp.int32)
        acc = (zero, zero, zero, zero)
        for c in range(n_chunks):
            slot = c % 2
            if c + 1 < n_chunks:
                for cp in copies(c + 1, (c + 1) % 2):
                    cp.start()
            for cp in copies(c, slot):
                cp.wait()

            def group_body(g, gcarry, slot=slot):
                tn0, tn1, st0, st1 = gcarry
                for u in range(_UNROLL):
                    off = g * (_L * _UNROLL) + u * _L
                    vx = xbuf[slot, pl.ds(off, _L)]
                    vt = tbuf[slot, pl.ds(off, _L)]
                    m = (vx < 0.5) & (vt == 0)
                    inc = jnp.where(m, 1, 0)
                    if u % 2 == 0:
                        tn0, st0 = tn0 + inc, st0 + vt
                    else:
                        tn1, st1 = tn1 + inc, st1 + vt
                return tn0, tn1, st0, st1

            acc = lax.fori_loop(0, groups // _UNROLL, group_body, acc)

        accbuf[0, :] = acc[0] + acc[1]
        accbuf[1, :] = acc[2] + acc[3]
        pltpu.sync_copy(accbuf, out_hbm.at[wid])

    return sc_count


def kernel(inputs, targets):
    n = inputs.shape[0]
    parts = _make_sc_count(n)(inputs, targets)
    tn = parts[:, 0, :].sum()
    st = parts[:, 1, :].sum()
    t0 = n - st
    return tn.astype(jnp.float32) / jnp.clip(t0.astype(jnp.float32), 1e-12)
